# baseline (speedup 1.0000x reference)
import functools

import jax
import jax.numpy as jnp
import numpy as np
from jax import lax
from jax.experimental import pallas as pl
from jax.experimental.pallas import tpu as pltpu
from jax.experimental.pallas import tpu_sc as plsc

N = 10000
E = 320000
IN_DIM = 128
HEADS = 8
HID = 16
HD = HEADS * HID
AW = HD + 16
NEG_SLOPE = 0.01

NC = 2
NS = 16
NW = NC * NS
LANES = 16

_TC_BLK = 400


def _pre_kernel(x_ref, wp_ref, ac_ref, ws_ref, h_ref, scat_ref, skip_ref):
    x = x_ref[...]
    h = jnp.dot(x, wp_ref[...], preferred_element_type=jnp.float32)
    h_ref[...] = h
    scat_ref[...] = jnp.dot(h, ac_ref[...], preferred_element_type=jnp.float32)
    skip_ref[...] = jnp.dot(x, ws_ref[...], preferred_element_type=jnp.float32)


def _merge_kernel(a0_ref, a1_ref, skip_ref, e8_ref, out_ref):
    a = a0_ref[0] + a1_ref[0]
    u = a[:, :HD]
    p = a[:, HD:HD + HEADS]
    r = 1.0 / (p + 1e-16)
    r128 = jnp.dot(r, e8_ref[...], preferred_element_type=jnp.float32)
    out_ref[...] = u * r128 + skip_ref[...]


def _make_sc_kernel(n_nodes, n_edges, blk):
    epw = n_edges // NW
    n_chunks = epw // blk
    rows_per_tile = n_nodes // NS
    n_win = pl.cdiv(n_nodes, blk)
    mesh = plsc.VectorSubcoreMesh(core_axis_name="c", subcore_axis_name="s")

    dnums = lax.GatherDimensionNumbers(
        offset_dims=(), collapsed_slice_dims=(0,), start_index_map=(0,))

    def lane_gather(v, idx):
        return lax.gather(v, idx.reshape(LANES, 1), dnums, (1,),
                          mode=lax.GatherScatterMode.PROMISE_IN_BOUNDS)

    @functools.partial(
        pl.kernel,
        out_type=jax.ShapeDtypeStruct((NC, n_nodes, AW), jnp.float32),
        mesh=mesh,
        compiler_params=pltpu.CompilerParams(use_tc_tiling_on_sc=False),
        scratch_types=[
            pltpu.VMEM_SHARED((n_nodes, AW), jnp.float32),
            pltpu.VMEM((blk,), jnp.int32),
            pltpu.VMEM((blk,), jnp.int32),
            pltpu.VMEM((blk, LANES), jnp.float32),
            pltpu.VMEM((blk, LANES), jnp.float32),
            pltpu.VMEM((blk, HD), jnp.float32),
            pltpu.VMEM((blk, AW), jnp.float32),
            pltpu.VMEM((blk,), jnp.int32),
            pltpu.VMEM((blk,), jnp.int32),
            pltpu.VMEM((blk,), jnp.int32),
            pltpu.VMEM((blk, LANES), jnp.float32),
            pltpu.VMEM((blk, LANES), jnp.float32),
            pltpu.VMEM((blk, HD), jnp.float32),
            pltpu.VMEM((blk, AW), jnp.float32),
            pltpu.VMEM((blk,), jnp.int32),
            pltpu.SemaphoreType.DMA,
            pltpu.SemaphoreType.DMA,
            pltpu.SemaphoreType.DMA,
            pltpu.SemaphoreType.DMA,
            pltpu.SemaphoreType.DMA,
            pltpu.SemaphoreType.DMA,
            pltpu.SemaphoreType.DMA,
        ],
    )
    def sc_edge_pass(scat_hbm, h_hbm, src_hbm, dst_hbm, out_hbm,
                     acc, srcb0, dstb0, sbuf0, dbuf0, hbuf0, stage0, sdst0,
                     srcb1, dstb1, sbuf1, dbuf1, hbuf1, stage1, sdst1,
                     sem_i0, sem_i1, sem_g0, sem_g1, sem_s0, sem_s1, sem_z):
        cid = lax.axis_index("c")
        sid = lax.axis_index("s")
        wid = cid * NS + sid

        lane_iota = lax.iota(jnp.int32, LANES)
        shift_idx = (lane_iota & 7) + 8
        head_idx = [jnp.full((LANES,), hh, jnp.int32) for hh in range(HEADS)]
        zero16 = jnp.zeros((LANES,), jnp.float32)

        sets = ((srcb0, dstb0, sbuf0, dbuf0, hbuf0, stage0, sdst0,
                 sem_i0, sem_g0, sem_s0),
                (srcb1, dstb1, sbuf1, dbuf1, hbuf1, stage1, sdst1,
                 sem_i1, sem_g1, sem_s1))

        @pl.loop(0, blk)
        def _(j):
            for col in range(AW // LANES):
                stage0[j, pl.ds(col * LANES, LANES)] = zero16

        nz = pl.cdiv(n_win, NS)
        for k in range(nz):
            w = sid + NS * k

            @pl.when(w < n_win)
            def _():
                pltpu.async_copy(stage0, acc.at[pl.ds(w * blk, blk)], sem_z)

        for k in range(nz):
            w = sid + NS * k

            @pl.when(w < n_win)
            def _():
                pltpu.make_async_copy(
                    stage0, acc.at[pl.ds(w * blk, blk)], sem_z).wait()

        plsc.subcore_barrier()

        def issue_idx(c, bs):
            base = wid * epw + c * blk
            pltpu.async_copy(src_hbm.at[pl.ds(base, blk)], bs[0], bs[7])
            pltpu.async_copy(dst_hbm.at[pl.ds(base, blk)], bs[1], bs[7])

        def wait_idx(c, bs):
            base = wid * epw + c * blk
            pltpu.make_async_copy(src_hbm.at[pl.ds(base, blk)], bs[0], bs[7]).wait()
            pltpu.make_async_copy(dst_hbm.at[pl.ds(base, blk)], bs[1], bs[7]).wait()

        def issue_gather(bs):
            pltpu.async_copy(scat_hbm.at[bs[0]], bs[2], bs[8])
            pltpu.async_copy(scat_hbm.at[bs[1]], bs[3], bs[8])
            pltpu.async_copy(h_hbm.at[bs[0]], bs[4], bs[8])

        def wait_gather(bs):
            pltpu.make_async_copy(scat_hbm.at[bs[0]], bs[2], bs[8]).wait()
            pltpu.make_async_copy(scat_hbm.at[bs[1]], bs[3], bs[8]).wait()
            pltpu.make_async_copy(h_hbm.at[bs[0]], bs[4], bs[8]).wait()

        def compute(bs):
            sb, db, hb, st = bs[2], bs[3], bs[4], bs[5]

            @plsc.parallel_loop(0, blk, unroll=4)
            def _(j):
                rs = sb[j, :]
                rd = db[j, :]
                sd = lane_gather(rd, shift_idx)
                e = rs + sd
                e = jnp.where(e >= 0.0, e, e * NEG_SLOPE)
                p = jnp.exp(e)
                st[j, pl.ds(HD, LANES)] = p
                for hh in range(HEADS):
                    pv = lane_gather(p, head_idx[hh])
                    hv = hb[j, pl.ds(hh * LANES, LANES)]
                    st[j, pl.ds(hh * LANES, LANES)] = hv * pv

        def issue_scatter(bs):
            pltpu.async_copy(bs[5], acc.at[bs[6]], bs[9], add=True)

        def wait_scatter(bs):
            pltpu.make_async_copy(bs[5], acc.at[bs[6]], bs[9]).wait()

        def body(cc, k):
            bs = sets[k]
            other = sets[1 - k]

            @pl.when(cc + 1 < n_chunks)
            def _():
                wait_idx(cc + 1, other)
                issue_gather(other)

            wait_gather(bs)

            @pl.when(cc >= 2)
            def _():
                wait_scatter(bs)

            for off in (0, 16, blk - LANES):
                bs[6][pl.ds(off, LANES)] = bs[1][pl.ds(off, LANES)]
            compute(bs)
            issue_scatter(bs)

            @pl.when(cc + 2 < n_chunks)
            def _():
                issue_idx(cc + 2, bs)

        assert n_chunks % 2 == 0
        issue_idx(0, sets[0])
        wait_idx(0, sets[0])
        issue_gather(sets[0])
        issue_idx(1, sets[1])

        @pl.loop(0, n_chunks, step=2)
        def _(c):
            body(c, 0)
            body(c + 1, 1)

        for k in range(2):
            wait_scatter(sets[k])

        plsc.subcore_barrier()

        r0 = sid * rows_per_tile
        pltpu.sync_copy(acc.at[pl.ds(r0, rows_per_tile)],
                        out_hbm.at[cid].at[pl.ds(r0, rows_per_tile)])

    return sc_edge_pass


_SC_BLK = 40


def kernel(x, edge_index, W_proj, att_e, W_skip):
    num_nodes = x.shape[0]

    wp_t = W_proj.T
    ws_t = W_skip.T
    a_src = att_e[0, :, :HID]
    a_dst = att_e[0, :, HID:]
    eye8 = jnp.eye(HEADS, dtype=jnp.float32)
    a_cat = jnp.concatenate(
        [
            (eye8[:, None, :] * a_src[:, :, None]).reshape(HD, HEADS),
            (eye8[:, None, :] * a_dst[:, :, None]).reshape(HD, HEADS),
        ],
        axis=1,
    )

    h, scat, skip = pl.pallas_call(
        _pre_kernel,
        grid=(num_nodes // _TC_BLK,),
        in_specs=[
            pl.BlockSpec((_TC_BLK, IN_DIM), lambda i: (i, 0)),
            pl.BlockSpec((IN_DIM, HD), lambda i: (0, 0)),
            pl.BlockSpec((HD, 2 * HEADS), lambda i: (0, 0)),
            pl.BlockSpec((IN_DIM, HD), lambda i: (0, 0)),
        ],
        out_specs=[
            pl.BlockSpec((_TC_BLK, HD), lambda i: (i, 0)),
            pl.BlockSpec((_TC_BLK, 2 * HEADS), lambda i: (i, 0)),
            pl.BlockSpec((_TC_BLK, HD), lambda i: (i, 0)),
        ],
        out_shape=[
            jax.ShapeDtypeStruct((num_nodes, HD), jnp.float32),
            jax.ShapeDtypeStruct((num_nodes, 2 * HEADS), jnp.float32),
            jax.ShapeDtypeStruct((num_nodes, HD), jnp.float32),
        ],
    )(x, wp_t, a_cat, ws_t)

    sc = _make_sc_kernel(num_nodes, E, _SC_BLK)
    acc = sc(scat, h, edge_index[0], edge_index[1])

    e8 = jnp.asarray(np.repeat(np.eye(HEADS, dtype=np.float32), HID, axis=1))

    out = pl.pallas_call(
        _merge_kernel,
        grid=(num_nodes // _TC_BLK,),
        in_specs=[
            pl.BlockSpec((1, _TC_BLK, AW), lambda i: (0, i, 0)),
            pl.BlockSpec((1, _TC_BLK, AW), lambda i: (1, i, 0)),
            pl.BlockSpec((_TC_BLK, HD), lambda i: (i, 0)),
            pl.BlockSpec((HEADS, HD), lambda i: (0, 0)),
        ],
        out_specs=pl.BlockSpec((_TC_BLK, HD), lambda i: (i, 0)),
        out_shape=jax.ShapeDtypeStruct((num_nodes, HD), jnp.float32),
    )(acc, acc, skip, e8)

    return out


# speedup vs baseline: 1.4510x; 1.4510x over previous
import functools

import jax
import jax.numpy as jnp
import numpy as np
from jax import lax
from jax.experimental import pallas as pl
from jax.experimental.pallas import tpu as pltpu
from jax.experimental.pallas import tpu_sc as plsc

N = 10000
E = 320000
IN_DIM = 128
HEADS = 8
HID = 16
HD = HEADS * HID
AW = HD + 16
NEG_SLOPE = 0.01

NC = 2
NS = 16
NW = NC * NS
LANES = 16

_TC_BLK = 2000


def _pre_kernel(x_ref, wp_ref, wpp_ref, ac_ref, ws_ref,
                scat_ref, hp_ref, skip_ref):
    x = x_ref[...]
    h = jnp.dot(x, wp_ref[...], preferred_element_type=jnp.float32)
    scat_ref[...] = jnp.dot(h, ac_ref[...], preferred_element_type=jnp.float32)
    hp = jnp.dot(x, wpp_ref[...], preferred_element_type=jnp.float32)
    hp_ref[...] = hp.astype(jnp.bfloat16)
    skip_ref[...] = jnp.dot(x, ws_ref[...], preferred_element_type=jnp.float32)


def _merge_kernel(a0_ref, a1_ref, skip_ref, e8_ref, out_ref):
    a = a0_ref[0] + a1_ref[0]
    u = a[:, :HD]
    p = a[:, HD:HD + HEADS]
    r = 1.0 / (p + 1e-16)
    r128 = jnp.dot(r, e8_ref[...], preferred_element_type=jnp.float32)
    out_ref[...] = u * r128 + skip_ref[...]


def _make_sc_kernel(n_nodes, n_edges, blk):
    epw = n_edges // NW
    n_chunks = epw // blk
    rows_per_tile = n_nodes // NS
    n_win = pl.cdiv(n_nodes, blk)
    mesh = plsc.VectorSubcoreMesh(core_axis_name="c", subcore_axis_name="s")

    dnums = lax.GatherDimensionNumbers(
        offset_dims=(), collapsed_slice_dims=(0,), start_index_map=(0,))

    def lane_gather(v, idx):
        return lax.gather(v, idx.reshape(LANES, 1), dnums, (1,),
                          mode=lax.GatherScatterMode.PROMISE_IN_BOUNDS)

    @functools.partial(
        pl.kernel,
        out_type=jax.ShapeDtypeStruct((NC, n_nodes, AW), jnp.float32),
        mesh=mesh,
        compiler_params=pltpu.CompilerParams(
            use_tc_tiling_on_sc=False, needs_layout_passes=False),
        scratch_types=[
            pltpu.VMEM_SHARED((n_nodes, AW), jnp.float32),
            pltpu.VMEM((blk,), jnp.int32),
            pltpu.VMEM((blk,), jnp.int32),
            pltpu.VMEM((blk, LANES), jnp.float32),
            pltpu.VMEM((blk, LANES), jnp.float32),
            pltpu.VMEM((blk, HD), jnp.bfloat16),
            pltpu.VMEM((blk, AW), jnp.float32),
            pltpu.VMEM((blk,), jnp.int32),
            pltpu.VMEM((blk,), jnp.int32),
            pltpu.VMEM((blk,), jnp.int32),
            pltpu.VMEM((blk, LANES), jnp.float32),
            pltpu.VMEM((blk, LANES), jnp.float32),
            pltpu.VMEM((blk, HD), jnp.bfloat16),
            pltpu.VMEM((blk, AW), jnp.float32),
            pltpu.VMEM((blk,), jnp.int32),
            pltpu.SemaphoreType.DMA,
            pltpu.SemaphoreType.DMA,
            pltpu.SemaphoreType.DMA,
            pltpu.SemaphoreType.DMA,
            pltpu.SemaphoreType.DMA,
            pltpu.SemaphoreType.DMA,
            pltpu.SemaphoreType.DMA,
        ],
    )
    def sc_edge_pass(scat_hbm, hp_hbm, ei_hbm, out_hbm,
                     acc, srcb0, dstb0, sbuf0, dbuf0, hbuf0, stage0, sdst0,
                     srcb1, dstb1, sbuf1, dbuf1, hbuf1, stage1, sdst1,
                     sem_i0, sem_i1, sem_g0, sem_g1, sem_s0, sem_s1, sem_z):
        cid = lax.axis_index("c")
        sid = lax.axis_index("s")
        wid = cid * NS + sid

        lane_iota = lax.iota(jnp.int32, LANES)
        shift_idx = (lane_iota & 7) + 8
        head_idx = [jnp.full((LANES,), hh, jnp.int32) for hh in range(HEADS)]
        zero16 = jnp.zeros((LANES,), jnp.float32)

        sets = ((srcb0, dstb0, sbuf0, dbuf0, hbuf0, stage0, sdst0,
                 sem_i0, sem_g0, sem_s0),
                (srcb1, dstb1, sbuf1, dbuf1, hbuf1, stage1, sdst1,
                 sem_i1, sem_g1, sem_s1))

        @pl.loop(0, blk)
        def _(j):
            for col in range(AW // LANES):
                stage0[j, pl.ds(col * LANES, LANES)] = zero16

        nz = pl.cdiv(n_win, NS)
        for k in range(nz):
            w = sid + NS * k

            @pl.when(w < n_win)
            def _():
                pltpu.async_copy(stage0, acc.at[pl.ds(w * blk, blk)], sem_z)

        for k in range(nz):
            w = sid + NS * k

            @pl.when(w < n_win)
            def _():
                pltpu.make_async_copy(
                    stage0, acc.at[pl.ds(w * blk, blk)], sem_z).wait()

        plsc.subcore_barrier()

        def issue_idx(c, bs):
            base = wid * epw + c * blk
            pltpu.async_copy(ei_hbm.at[0].at[pl.ds(base, blk)], bs[0], bs[7])
            pltpu.async_copy(ei_hbm.at[1].at[pl.ds(base, blk)], bs[1], bs[7])

        def wait_idx(c, bs):
            base = wid * epw + c * blk
            pltpu.make_async_copy(
                ei_hbm.at[0].at[pl.ds(base, blk)], bs[0], bs[7]).wait()
            pltpu.make_async_copy(
                ei_hbm.at[1].at[pl.ds(base, blk)], bs[1], bs[7]).wait()

        def issue_gather(bs):
            pltpu.async_copy(scat_hbm.at[bs[0]], bs[2], bs[8])
            pltpu.async_copy(scat_hbm.at[bs[1]], bs[3], bs[8])
            pltpu.async_copy(hp_hbm.at[bs[0]], bs[4], bs[8])

        def wait_gather(bs):
            pltpu.make_async_copy(scat_hbm.at[bs[0]], bs[2], bs[8]).wait()
            pltpu.make_async_copy(scat_hbm.at[bs[1]], bs[3], bs[8]).wait()
            pltpu.make_async_copy(hp_hbm.at[bs[0]], bs[4], bs[8]).wait()

        def compute(bs):
            sb, db, hb, st = bs[2], bs[3], bs[4], bs[5]

            @plsc.parallel_loop(0, blk, unroll=4)
            def _(j):
                rs = sb[j, :]
                rd = db[j, :]
                sd = lane_gather(rd, shift_idx)
                e = rs + sd
                e = jnp.where(e >= 0.0, e, e * NEG_SLOPE)
                p = jnp.exp(e)
                st[j, pl.ds(HD, LANES)] = p
                for a in range(HEADS // 2):
                    v = hb[j, pl.ds(32 * a, 32)]
                    va, vb = plsc.unpack(
                        v, format=plsc.PackFormat.INTERLEAVED,
                        preferred_element_type=jnp.float32)
                    pva = lane_gather(p, head_idx[2 * a])
                    pvb = lane_gather(p, head_idx[2 * a + 1])
                    st[j, pl.ds(32 * a, LANES)] = va * pva
                    st[j, pl.ds(32 * a + LANES, LANES)] = vb * pvb

        def issue_scatter(bs):
            pltpu.async_copy(bs[5], acc.at[bs[6]], bs[9], add=True)

        def wait_scatter(bs):
            pltpu.make_async_copy(bs[5], acc.at[bs[6]], bs[9]).wait()

        def body(cc, k):
            bs = sets[k]
            other = sets[1 - k]

            @pl.when(cc + 1 < n_chunks)
            def _():
                wait_idx(cc + 1, other)
                issue_gather(other)

            wait_gather(bs)

            @pl.when(cc >= 2)
            def _():
                wait_scatter(bs)

            for off in range(0, blk, LANES):
                bs[6][pl.ds(off, LANES)] = bs[1][pl.ds(off, LANES)]
            compute(bs)
            issue_scatter(bs)

            @pl.when(cc + 2 < n_chunks)
            def _():
                issue_idx(cc + 2, bs)

        assert n_chunks % 2 == 1
        issue_idx(0, sets[0])
        wait_idx(0, sets[0])
        issue_gather(sets[0])
        issue_idx(1, sets[1])

        @pl.loop(0, n_chunks - 1, step=2)
        def _(c):
            body(c, 0)
            body(c + 1, 1)

        body(jnp.int32(n_chunks - 1), 0)

        for k in range(2):
            wait_scatter(sets[k])

        plsc.subcore_barrier()

        r0 = sid * rows_per_tile
        pltpu.sync_copy(acc.at[pl.ds(r0, rows_per_tile)],
                        out_hbm.at[cid].at[pl.ds(r0, rows_per_tile)])

    return sc_edge_pass


_SC_BLK = 80


def _perm_matrix():
    perm = np.zeros((HD, HD), np.float32)
    for a in range(HEADS // 2):
        for i in range(HID):
            perm[32 * a + i, 32 * a + 2 * i] = 1.0
            perm[32 * a + 16 + i, 32 * a + 2 * i + 1] = 1.0
    return perm


def kernel(x, edge_index, W_proj, att_e, W_skip):
    num_nodes = x.shape[0]

    wp_t = W_proj.T
    ws_t = W_skip.T
    wpp = wp_t @ jnp.asarray(_perm_matrix())
    a_src = att_e[0, :, :HID]
    a_dst = att_e[0, :, HID:]
    eye8 = jnp.eye(HEADS, dtype=jnp.float32)
    a_cat = jnp.concatenate(
        [
            (eye8[:, None, :] * a_src[:, :, None]).reshape(HD, HEADS),
            (eye8[:, None, :] * a_dst[:, :, None]).reshape(HD, HEADS),
        ],
        axis=1,
    )

    scat, hp, skip = pl.pallas_call(
        _pre_kernel,
        grid=(num_nodes // _TC_BLK,),
        in_specs=[
            pl.BlockSpec((_TC_BLK, IN_DIM), lambda i: (i, 0)),
            pl.BlockSpec((IN_DIM, HD), lambda i: (0, 0)),
            pl.BlockSpec((IN_DIM, HD), lambda i: (0, 0)),
            pl.BlockSpec((HD, 2 * HEADS), lambda i: (0, 0)),
            pl.BlockSpec((IN_DIM, HD), lambda i: (0, 0)),
        ],
        out_specs=[
            pl.BlockSpec((_TC_BLK, 2 * HEADS), lambda i: (i, 0)),
            pl.BlockSpec((_TC_BLK, HD), lambda i: (i, 0)),
            pl.BlockSpec((_TC_BLK, HD), lambda i: (i, 0)),
        ],
        out_shape=[
            jax.ShapeDtypeStruct((num_nodes, 2 * HEADS), jnp.float32),
            jax.ShapeDtypeStruct((num_nodes, HD), jnp.bfloat16),
            jax.ShapeDtypeStruct((num_nodes, HD), jnp.float32),
        ],
    )(x, wp_t, wpp, a_cat, ws_t)

    sc = _make_sc_kernel(num_nodes, E, _SC_BLK)
    acc = sc(scat, hp, edge_index)

    e8 = jnp.asarray(np.repeat(np.eye(HEADS, dtype=np.float32), HID, axis=1))

    out = pl.pallas_call(
        _merge_kernel,
        grid=(num_nodes // _TC_BLK,),
        in_specs=[
            pl.BlockSpec((1, _TC_BLK, AW), lambda i: (0, i, 0)),
            pl.BlockSpec((1, _TC_BLK, AW), lambda i: (1, i, 0)),
            pl.BlockSpec((_TC_BLK, HD), lambda i: (i, 0)),
            pl.BlockSpec((HEADS, HD), lambda i: (0, 0)),
        ],
        out_specs=pl.BlockSpec((_TC_BLK, HD), lambda i: (i, 0)),
        out_shape=jax.ShapeDtypeStruct((num_nodes, HD), jnp.float32),
    )(acc, acc, skip, e8)

    return out
